# baseline (device time: 11231 ns/iter reference)
import jax
import jax.numpy as jnp
from jax import lax
from jax.experimental import pallas as pl
from jax.experimental.pallas import tpu as pltpu

N_DEV = 16
PLANE = 4


def kernel(x):
    m, n = x.shape

    def body(x_ref, out_ref, xv_ref, c1_ref, c2_ref, copy_sem,
             send1, recv1, send2, recv2):
        my = lax.axis_index("i")
        z = lax.div(my, PLANE)
        s = lax.rem(my, PLANE)

        def plane_peer(dt):
            return z * PLANE + lax.rem(s + dt, PLANE)

        def column_peer(dz):
            return lax.rem(z + dz, PLANE) * PLANE + s

        barrier_sem = pltpu.get_barrier_semaphore()
        for dt in range(1, PLANE):
            pl.semaphore_signal(
                barrier_sem, inc=5,
                device_id=(plane_peer(dt),),
                device_id_type=pl.DeviceIdType.MESH,
            )
        for dz in range(1, PLANE):
            pl.semaphore_signal(
                barrier_sem, inc=1,
                device_id=(column_peer(dz),),
                device_id_type=pl.DeviceIdType.MESH,
            )

        in_copy = pltpu.make_async_copy(x_ref, xv_ref, copy_sem)
        in_copy.start()
        in_copy.wait()
        c1_ref[0, :, :] = jnp.max(xv_ref[...], axis=0, keepdims=True)

        pl.semaphore_wait(barrier_sem, 15)

        rdmas1 = []
        for dt in range(1, PLANE):
            rdma = pltpu.make_async_remote_copy(
                src_ref=c1_ref.at[0],
                dst_ref=c1_ref.at[dt],
                send_sem=send1.at[dt],
                recv_sem=recv1.at[dt],
                device_id=(plane_peer(dt),),
                device_id_type=pl.DeviceIdType.MESH,
            )
            rdma.start()
            rdmas1.append(rdma)
        for rdma in rdmas1:
            rdma.wait_recv()

        c2_ref[0, :, :] = jnp.max(c1_ref[...], axis=0)

        pl.semaphore_wait(barrier_sem, 3)

        rdmas2 = []
        for dz in range(1, PLANE):
            rdma = pltpu.make_async_remote_copy(
                src_ref=c2_ref.at[0],
                dst_ref=c2_ref.at[dz],
                send_sem=send2.at[dz],
                recv_sem=recv2.at[dz],
                device_id=(column_peer(dz),),
                device_id_type=pl.DeviceIdType.MESH,
            )
            rdma.start()
            rdmas2.append(rdma)
        for rdma in rdmas2:
            rdma.wait_recv()

        out_ref[...] = jnp.max(c2_ref[...], axis=0)

        for rdma in rdmas1 + rdmas2:
            rdma.wait_send()

    return pl.pallas_call(
        body,
        out_shape=jax.ShapeDtypeStruct((1, n), x.dtype),
        in_specs=[pl.BlockSpec(memory_space=pltpu.MemorySpace.HBM)],
        out_specs=pl.BlockSpec(memory_space=pltpu.VMEM),
        scratch_shapes=[
            pltpu.VMEM((m, n), x.dtype),
            pltpu.VMEM((PLANE, 1, n), x.dtype),
            pltpu.VMEM((PLANE, 1, n), x.dtype),
            pltpu.SemaphoreType.DMA,
            pltpu.SemaphoreType.DMA((PLANE,)),
            pltpu.SemaphoreType.DMA((PLANE,)),
            pltpu.SemaphoreType.DMA((PLANE,)),
            pltpu.SemaphoreType.DMA((PLANE,)),
        ],
        compiler_params=pltpu.CompilerParams(collective_id=0),
    )(x)
